# baseline (speedup 1.0000x reference)
import jax
import jax.numpy as jnp
from jax.experimental import pallas as pl
from jax.experimental.pallas import tpu as pltpu


def _body(x_hbm, cb_hbm, out_ref, x_v, cb_v, sem_x, sem_cb):
    M, D = x_v.shape
    K = cb_v.shape[0]
    cpx = pltpu.make_async_copy(x_hbm, x_v, sem_x)
    cpc = pltpu.make_async_copy(cb_hbm, cb_v, sem_cb)
    cpx.start()
    cpc.start()
    cpx.wait()
    xv = x_v[...]
    x2 = jnp.sum(xv * xv, axis=1, keepdims=True)
    cpc.wait()
    cb = cb_v[...]
    c2 = jnp.sum(cb * cb, axis=1)[None, :]
    xc = jax.lax.dot_general(xv, cb, (((1,), (1,)), ((), ())),
                             preferred_element_type=jnp.float32)
    dist = x2 + c2 - 2.0 * xc
    mins = jnp.min(dist, axis=1, keepdims=True)
    kio = jax.lax.broadcasted_iota(jnp.int32, (M, K), 1).astype(jnp.float32)
    idx = jnp.min(jnp.where(dist == mins, kio, jnp.float32(K)),
                  axis=1, keepdims=True)
    onehot = jnp.where(kio == idx, jnp.float32(1), jnp.float32(0))
    out_ref[...] = jax.lax.dot_general(
        onehot, cb, (((1,), (0,)), ((), ())),
        preferred_element_type=jnp.float32)


def kernel(x, codebook):
    N, C, H, W = x.shape
    K = codebook.shape[0]
    D = H * W
    M = N * C
    x_flat = x.reshape(M, D)
    cb_flat = codebook.reshape(K, D)
    out = pl.pallas_call(
        _body,
        in_specs=[pl.BlockSpec(memory_space=pl.ANY),
                  pl.BlockSpec(memory_space=pl.ANY)],
        out_shape=jax.ShapeDtypeStruct((M, D), jnp.float32),
        scratch_shapes=[
            pltpu.VMEM((M, D), jnp.float32),
            pltpu.VMEM((K, D), jnp.float32),
            pltpu.SemaphoreType.DMA,
            pltpu.SemaphoreType.DMA,
        ],
    )(x_flat, cb_flat)
    return out.reshape(N, C, H, W)
